# Initial kernel scaffold; baseline (speedup 1.0000x reference)
import jax
import jax.numpy as jnp
from jax.experimental import pallas as pl


def kernel(z, W):
    raise NotImplementedError("write your pallas kernel here")


# speedup vs baseline: 1.2325x; 1.2325x over previous
import functools

import jax
import jax.numpy as jnp
from jax import lax
from jax.experimental import pallas as pl
from jax.experimental.pallas import tpu as pltpu
from jax.experimental.pallas import tpu_sc as plsc

N_EMBED = 8192
EMBED_DIM = 32
ROWS = 1024
CODE_CHUNK = 1024
N_CHUNKS = N_EMBED // CODE_CHUNK

NC, NS = 2, 16
NW = NC * NS
B_TOT = 16 * 32 * 32
B_PER_W = B_TOT // NW
GCHUNK = 128
NGC = B_PER_W // GCHUNK
GDIM = 128


def _vq_tc_kernel(z_ref, w_ref, zs_ref, ws_ref, idx_ref, lsum_ref):
    z = z_ref[0].T
    rowz = zs_ref[0].T
    strip_min = []
    strip_arg = []
    for c in range(N_CHUNKS):
        wc = w_ref[pl.ds(c * CODE_CHUNK, CODE_CHUNK), :]
        roww = ws_ref[0, pl.ds(c * CODE_CHUNK, CODE_CHUNK)]
        mm = lax.dot_general(z, wc, (((1,), (1,)), ((), ())),
                             preferred_element_type=jnp.float32)
        d = (rowz + roww[None, :]) - 2.0 * mm
        cmin = jnp.min(d, axis=1, keepdims=True)
        ii = lax.broadcasted_iota(jnp.int32, d.shape, 1)
        carg = jnp.min(jnp.where(d == cmin, ii, N_EMBED), axis=1,
                       keepdims=True) + c * CODE_CHUNK
        if c % 4 == 0:
            strip_min.append(cmin)
            strip_arg.append(carg)
        else:
            take = cmin < strip_min[-1]
            strip_min[-1] = jnp.where(take, cmin, strip_min[-1])
            strip_arg[-1] = jnp.where(take, carg, strip_arg[-1])
    acc_b = strip_min[0].astype(jnp.bfloat16).astype(jnp.float32)
    acc_v = strip_min[0]
    bidx = strip_arg[0]
    for s in range(1, len(strip_min)):
        take = strip_min[s] < acc_b
        acc_b = jnp.where(take,
                          strip_min[s].astype(jnp.bfloat16).astype(jnp.float32),
                          acc_b)
        acc_v = jnp.where(take, strip_min[s], acc_v)
        bidx = jnp.where(take, strip_arg[s], bidx)
    idx_ref[0] = bidx.T
    bsum = jnp.sum(acc_v)
    i = pl.program_id(0)
    prev = jnp.where(i == 0, 0.0, lsum_ref[0, 0])
    lsum_ref[0, 0] = prev + bsum


def _tc_distance_argmin(zf, w, zsum, wsum):
    idx, lsum = pl.pallas_call(
        _vq_tc_kernel,
        grid=(16,),
        in_specs=[
            pl.BlockSpec((1, EMBED_DIM, ROWS), lambda i: (i, 0, 0)),
            pl.BlockSpec((N_EMBED, EMBED_DIM), lambda i: (0, 0)),
            pl.BlockSpec((1, 1, ROWS), lambda i: (i, 0, 0)),
            pl.BlockSpec((1, N_EMBED), lambda i: (0, 0)),
        ],
        out_specs=[
            pl.BlockSpec((1, 1, ROWS), lambda i: (i, 0, 0)),
            pl.BlockSpec((1, 1), lambda i: (0, 0), memory_space=pltpu.SMEM),
        ],
        out_shape=[
            jax.ShapeDtypeStruct((16, 1, ROWS), jnp.int32),
            jax.ShapeDtypeStruct((1, 1), jnp.float32),
        ],
    )(zf, w, zsum, wsum)
    return idx, lsum


def _sc_gather_build():
    mesh = plsc.VectorSubcoreMesh(core_axis_name="c", subcore_axis_name="s")

    @functools.partial(
        pl.kernel,
        mesh=mesh,
        out_type=jax.ShapeDtypeStruct((B_TOT, GDIM), jnp.float32),
        scratch_types=[
            pltpu.VMEM((NGC, GCHUNK), jnp.int32),
            pltpu.VMEM((B_PER_W, GDIM), jnp.float32),
            pltpu.SemaphoreType.DMA,
        ],
    )
    def gather(w_hbm, idx_hbm, out_hbm, idx_v, rows_v, sem):
        wid = lax.axis_index("s") * NC + lax.axis_index("c")
        pltpu.sync_copy(idx_hbm.at[wid], idx_v)
        for j in range(NGC):
            pltpu.async_copy(w_hbm.at[idx_v.at[j]],
                             rows_v.at[pl.ds(j * GCHUNK, GCHUNK)], sem).wait()
        pltpu.sync_copy(rows_v, out_hbm.at[pl.ds(wid * B_PER_W, B_PER_W)])

    return gather


def kernel(z, W):
    B, C, H, Wsp = z.shape
    zf = z.reshape(B, C, H * Wsp)
    zp4 = jnp.transpose(z, (0, 2, 3, 1))
    zsum = jnp.sum(zp4.reshape(B * H * Wsp, C) ** 2, axis=1).reshape(B, 1, H * Wsp)
    wsum = jnp.sum(W ** 2, axis=1).reshape(1, N_EMBED)
    idx, lsum = _tc_distance_argmin(zf, W, zsum, wsum)
    idx_w = idx.reshape(NW, NGC, GCHUNK)
    w_pad = jnp.pad(W, ((0, 0), (0, GDIM - EMBED_DIM)))
    zq_flat = _sc_gather_build()(w_pad, idx_w)[:, :EMBED_DIM]
    z_q = zp4 + (zq_flat.reshape(B, H, Wsp, C) - zp4)
    z_q = z_q.transpose(0, 3, 1, 2)
    loss = 2.0 * lsum[0, 0] / (B * C * H * Wsp)
    return (z_q, loss)
